# baseline (speedup 1.0000x reference)
import jax
import jax.numpy as jnp
from jax import lax
from jax.experimental import pallas as pl
from jax.experimental.pallas import tpu as pltpu
from jax.experimental.pallas import tpu_sc as plsc

H = 512
W = 512
NPOS = 11
BW = 5
SH = 3
NSEG = 8
L16 = 16
NITEMS = 64
INF = float("inf")


def _vgather(x, idx):
    dnums = lax.GatherDimensionNumbers(
        offset_dims=(), collapsed_slice_dims=(0,), start_index_map=(0,))
    return lax.gather(x, idx[:, None], dnums, (1,),
                      mode=lax.GatherScatterMode.PROMISE_IN_BOUNDS)


def _sc_dp_body(gm_hbm, paths_hbm, bandv2, bandh2, bandt, path_v, cost_v,
                outp_v, sem):
    del sem
    cid = lax.axis_index("c")
    sid = lax.axis_index("s")
    wid = sid * 2 + cid
    iota = lax.broadcasted_iota(jnp.int32, (L16,), 0)
    shl = jnp.maximum(iota - 1, 0)
    shr = jnp.minimum(iota + 1, L16 - 1)
    zero16 = jnp.zeros((L16,), jnp.int32)
    in_band = (iota >= SH) & (iota < SH + NPOS)
    b_img = wid // 8
    p_slot = lax.rem(wid, 8)
    HP = H + 1

    def run_dp(item, abase, cost0, load_row):
        @plsc.parallel_loop(1, H, carry=cost0, unroll=4)
        def fwd(l, cost):
            a = jnp.where(iota == SH, INF, _vgather(cost, shl))
            c = _vgather(cost, shr)
            m = jnp.minimum(jnp.minimum(a, cost), c)
            take_l = (a <= cost) & (a <= c)
            take_m = cost <= c
            delta = jnp.where(take_l, -1, jnp.where(take_m, 0, 1))
            path_v[pl.ds(l * L16, L16)] = iota + delta.astype(jnp.int32)
            return jnp.where(in_band, m - load_row(l), INF)

        cost_v[pl.ds(0, L16)] = fwd

        def amin(j, carry):
            best, bidx = carry
            c = cost_v[pl.ds(j, L16)][0]
            pred = c < best
            return (jnp.where(pred, c, best),
                    jnp.where(pred, j, bidx))

        _, idx0 = lax.fori_loop(SH, SH + NPOS, amin, (INF, jnp.int32(SH)))

        @plsc.parallel_loop(0, H, carry=(idx0, zero16), unroll=4)
        def bwd(t, carry):
            idx, acc = carry
            l = (H - 1) - t
            lane = lax.rem(l, L16)
            acc = jnp.where(iota == lane, abase + idx, acc)

            @pl.when(lane == 0)
            def _():
                outp_v[pl.ds(l, L16)] = acc

            nidx = path_v[pl.ds(l * L16 + idx, L16)][0]
            return (nidx, acc)
        pltpu.sync_copy(outp_v, paths_hbm.at[item])

    @pl.when(p_slot < 7)
    def _():
        abase = (p_slot + 1) * 64 - BW - SH
        item_v = b_img * 16 + p_slot
        with jax.named_scope("dma_v"):
            pltpu.sync_copy(gm_hbm.at[b_img, :, pl.ds(abase, L16)], bandv2)
        with jax.named_scope("dp_v"):
            cost0 = jnp.where(in_band, -bandv2[0], INF)
            run_dp(item_v, abase, cost0, lambda l: bandv2[l])

        item_h = b_img * 16 + 8 + p_slot
        with jax.named_scope("dma_h"):
            for k in range(L16):
                pltpu.sync_copy(gm_hbm.at[b_img, abase + k, :],
                                bandh2.at[pl.ds(k * H, H)])

        with jax.named_scope("relay"):
            @plsc.parallel_loop(0, H, unroll=8)
            def relay(t):
                bandt[pl.ds(t * L16 + t // 32, L16)] = bandh2[pl.ds(t * L16, L16)]

        with jax.named_scope("dp_h"):
            col0 = iota * HP
            cost0h = jnp.where(in_band, -plsc.load_gather(bandt, [col0]), INF)
            run_dp(item_h, abase, cost0h,
                   lambda l: plsc.load_gather(bandt, [col0 + l]))


def _label_body(paths_ref, out_ref):
    iw = lax.broadcasted_iota(jnp.int32, (H, W), 1)
    ih = lax.broadcasted_iota(jnp.int32, (H, W), 0)
    acc_v = jnp.zeros((H, W), jnp.int32)
    acc_h = jnp.zeros((H, W), jnp.int32)
    for p in range(7):
        vp = paths_ref[0, p, :]
        acc_v += (vp[:, None] <= iw).astype(jnp.int32)
    for q in range(7):
        hq = paths_ref[0, 8 + q, :]
        acc_h += (hq[None, :] <= ih).astype(jnp.int32)
    out_ref[0] = acc_v + NSEG * acc_h


@jax.jit
def _run(gm):
    B = gm.shape[0]
    mesh = plsc.VectorSubcoreMesh(
        core_axis_name="c", subcore_axis_name="s", num_cores=2,
        num_subcores=16)
    sc_call = pl.kernel(
        _sc_dp_body,
        out_type=jax.ShapeDtypeStruct((NITEMS, H), jnp.int32),
        mesh=mesh,
        scratch_types=[
            pltpu.VMEM((H, L16), jnp.float32),
            pltpu.VMEM((L16 * H,), jnp.float32),
            pltpu.VMEM((L16 * (H + 1),), jnp.float32),
            pltpu.VMEM((H * L16 + L16,), jnp.int32),
            pltpu.VMEM((32,), jnp.float32),
            pltpu.VMEM((H,), jnp.int32),
            pltpu.SemaphoreType.DMA,
        ],
        compiler_params=pltpu.CompilerParams(use_tc_tiling_on_sc=False,
                                             needs_layout_passes=False),
    )
    paths = sc_call(gm)
    paths = paths.reshape(B, 2 * 8, H)

    out = pl.pallas_call(
        _label_body,
        grid=(B,),
        in_specs=[pl.BlockSpec((1, 2 * 8, H), lambda b: (b, 0, 0))],
        out_specs=pl.BlockSpec((1, H, W), lambda b: (b, 0, 0)),
        out_shape=jax.ShapeDtypeStruct((B, H, W), jnp.int32),
    )(paths)
    return out


def kernel(grad_map, segmentation_mask, band_width):
    del segmentation_mask, band_width
    return _run(grad_map[:, 0])


# speedup vs baseline: 1.1435x; 1.1435x over previous
import jax
import jax.numpy as jnp
from jax import lax
from jax.experimental import pallas as pl
from jax.experimental.pallas import tpu as pltpu
from jax.experimental.pallas import tpu_sc as plsc

H = 512
W = 512
NPOS = 11
BW = 5
SH = 3
NSEG = 8
L16 = 16
NITEMS = 64
INF = float("inf")


def _vgather(x, idx):
    dnums = lax.GatherDimensionNumbers(
        offset_dims=(), collapsed_slice_dims=(0,), start_index_map=(0,))
    return lax.gather(x, idx[:, None], dnums, (1,),
                      mode=lax.GatherScatterMode.PROMISE_IN_BOUNDS)


def _sc_dp_body(gm_hbm, paths_hbm, bandv2, bandh2, bandt, path_v, cost_v,
                outp_v, semv, semh):
    cid = lax.axis_index("c")
    sid = lax.axis_index("s")
    wid = sid * 2 + cid
    iota = lax.broadcasted_iota(jnp.int32, (L16,), 0)
    shl = jnp.maximum(iota - 1, 0)
    shr = jnp.minimum(iota + 1, L16 - 1)
    zero16 = jnp.zeros((L16,), jnp.int32)
    in_band = (iota >= SH) & (iota < SH + NPOS)
    b_img = wid // 8
    p_slot = lax.rem(wid, 8)
    HP = H + 1

    def run_dp(slot, abase, cost0, load_row):
        @plsc.parallel_loop(1, H, carry=cost0, unroll=8)
        def fwd(l, cost):
            a = _vgather(cost, shl)
            c = _vgather(cost, shr)
            m = jnp.minimum(jnp.minimum(a, cost), c)
            take_l = (a <= cost) & (a <= c)
            take_m = cost <= c
            delta = jnp.where(take_l, -1, jnp.where(take_m, 0, 1))
            path_v[pl.ds(l * L16, L16)] = iota + delta.astype(jnp.int32)
            return jnp.where(in_band, m - load_row(l), INF)

        cost_v[pl.ds(0, L16)] = fwd

        def amin(j, carry):
            best, bidx = carry
            c = cost_v[pl.ds(j, L16)][0]
            pred = c < best
            return (jnp.where(pred, c, best),
                    jnp.where(pred, j, bidx))

        _, idx0 = lax.fori_loop(SH, SH + NPOS, amin, (INF, jnp.int32(SH)))

        @plsc.parallel_loop(0, H, carry=(idx0, zero16), unroll=8)
        def bwd(t, carry):
            idx, acc = carry
            l = (H - 1) - t
            lane = lax.rem(l, L16)
            acc = jnp.where(iota == lane, abase + idx, acc)

            @pl.when(lane == 0)
            def _():
                outp_v[pl.ds(l, L16)] = acc

            nidx = path_v[pl.ds(l * L16 + idx, L16)][0]
            return (nidx, acc)
        pltpu.sync_copy(outp_v, paths_hbm.at[b_img, slot])

    @pl.when(p_slot < 7)
    def _():
        abase = (p_slot + 1) * 64 - BW - SH
        hv = pltpu.async_copy(gm_hbm.at[b_img, :, pl.ds(abase, L16)],
                              bandv2, semv)
        hh = pltpu.async_copy(gm_hbm.at[b_img, pl.ds(abase, L16), :],
                              bandh2, semh)
        with jax.named_scope("dp_v"):
            hv.wait()
            cost0 = jnp.where(in_band, -bandv2[0], INF)
            run_dp(p_slot, abase, cost0, lambda l: bandv2[l])

        with jax.named_scope("relay"):
            hh.wait()

            @plsc.parallel_loop(0, H, unroll=8)
            def relay(t):
                bandt[pl.ds(t * L16 + t // 32, L16)] = \
                    bandh2[t // 32, pl.ds(lax.rem(t, 32) * L16, L16)]

        with jax.named_scope("dp_h"):
            col0 = iota * HP
            cost0h = jnp.where(in_band, -plsc.load_gather(bandt, [col0]), INF)
            run_dp(8 + p_slot, abase, cost0h,
                   lambda l: plsc.load_gather(bandt, [col0 + l]))


def _label_body(paths_ref, out_ref):
    iw = lax.broadcasted_iota(jnp.int32, (H, W), 1)
    ih = lax.broadcasted_iota(jnp.int32, (H, W), 0)
    acc_v = jnp.zeros((H, W), jnp.int32)
    acc_h = jnp.zeros((H, W), jnp.int32)
    for p in range(7):
        vp = paths_ref[0, p, :]
        acc_v += (vp[:, None] <= iw).astype(jnp.int32)
    for q in range(7):
        hq = paths_ref[0, 8 + q, :]
        acc_h += (hq[None, :] <= ih).astype(jnp.int32)
    out_ref[0] = acc_v + NSEG * acc_h


@jax.jit
def _run(gm):
    B = gm.shape[0]
    mesh = plsc.VectorSubcoreMesh(
        core_axis_name="c", subcore_axis_name="s", num_cores=2,
        num_subcores=16)
    sc_call = pl.kernel(
        _sc_dp_body,
        out_type=jax.ShapeDtypeStruct((4, L16, H), jnp.int32),
        mesh=mesh,
        scratch_types=[
            pltpu.VMEM((H, L16), jnp.float32),
            pltpu.VMEM((L16, H), jnp.float32),
            pltpu.VMEM((L16 * (H + 1),), jnp.float32),
            pltpu.VMEM((H * L16 + L16,), jnp.int32),
            pltpu.VMEM((32,), jnp.float32),
            pltpu.VMEM((H,), jnp.int32),
            pltpu.SemaphoreType.DMA,
            pltpu.SemaphoreType.DMA,
        ],
        compiler_params=pltpu.CompilerParams(use_tc_tiling_on_sc=False,
                                             needs_layout_passes=False),
    )
    paths = sc_call(gm)

    out = pl.pallas_call(
        _label_body,
        grid=(B,),
        in_specs=[pl.BlockSpec((1, 2 * 8, H), lambda b: (b, 0, 0))],
        out_specs=pl.BlockSpec((1, H, W), lambda b: (b, 0, 0)),
        out_shape=jax.ShapeDtypeStruct((B, H, W), jnp.int32),
    )(paths)
    return out


def kernel(grad_map, segmentation_mask, band_width):
    del segmentation_mask, band_width
    return _run(grad_map[:, 0])


# speedup vs baseline: 1.4307x; 1.2511x over previous
import jax
import jax.numpy as jnp
from jax import lax
from jax.experimental import pallas as pl
from jax.experimental.pallas import tpu as pltpu
from jax.experimental.pallas import tpu_sc as plsc

H = 512
W = 512
NPOS = 11
BW = 5
SH = 3
NSEG = 8
L16 = 16
NITEMS = 64
INF = float("inf")


def _vgather(x, idx):
    dnums = lax.GatherDimensionNumbers(
        offset_dims=(), collapsed_slice_dims=(0,), start_index_map=(0,))
    return lax.gather(x, idx[:, None], dnums, (1,),
                      mode=lax.GatherScatterMode.PROMISE_IN_BOUNDS)


def _sc_dp_body(gm_hbm, paths_hbm, bandv2, bandh2, bandt, patha, pathb,
                cost_v, outpa, outpb, semv, semh):
    cid = lax.axis_index("c")
    sid = lax.axis_index("s")
    wid = sid * 2 + cid
    iota = lax.broadcasted_iota(jnp.int32, (L16,), 0)
    shl = jnp.maximum(iota - 1, 0)
    shr = jnp.minimum(iota + 1, L16 - 1)
    zero16 = jnp.zeros((L16,), jnp.int32)
    in_band = (iota >= SH) & (iota < SH + NPOS)
    b_img = wid // 8
    p_slot = lax.rem(wid, 8)
    HP = H + 1

    def fused_dp():
        abase = (p_slot + 1) * 64 - BW - SH
        col0 = iota * HP
        cost0a = jnp.where(in_band, -bandv2[0], INF)
        cost0b = jnp.where(in_band, -plsc.load_gather(bandt, [col0]), INF)

        @plsc.parallel_loop(1, H, carry=(cost0a, cost0b), unroll=8)
        def fwd(l, carry):
            ca, cb = carry
            out = []
            for cost, path_ref in ((ca, patha), (cb, pathb)):
                a = _vgather(cost, shl)
                c = _vgather(cost, shr)
                m = jnp.minimum(jnp.minimum(a, cost), c)
                take_l = (a <= cost) & (a <= c)
                take_m = cost <= c
                delta = jnp.where(take_l, -1, jnp.where(take_m, 0, 1))
                path_ref[pl.ds(l * L16, L16)] = iota + delta.astype(jnp.int32)
                out.append(m)
            na = jnp.where(in_band, out[0] - bandv2[l], INF)
            nb = jnp.where(in_band,
                           out[1] - plsc.load_gather(bandt, [col0 + l]), INF)
            return (na, nb)

        fa, fb = fwd
        cost_v[pl.ds(0, L16)] = fa
        cost_v[pl.ds(L16, L16)] = fb

        def amin(j, carry):
            best, bidx = carry
            c = cost_v[pl.ds(j, L16)][0]
            pred = c < best
            return (jnp.where(pred, c, best),
                    jnp.where(pred, j, bidx))

        _, ia0 = lax.fori_loop(SH, SH + NPOS, amin, (INF, jnp.int32(SH)))
        _, ib0 = lax.fori_loop(L16 + SH, L16 + SH + NPOS, amin,
                               (INF, jnp.int32(L16 + SH)))
        ib0 = ib0 - L16

        @plsc.parallel_loop(0, H, carry=(ia0, zero16, ib0, zero16), unroll=8)
        def bwd(t, carry):
            ia, acca, ib, accb = carry
            l = (H - 1) - t
            lane = lax.rem(l, L16)
            acca = jnp.where(iota == lane, abase + ia, acca)
            accb = jnp.where(iota == lane, abase + ib, accb)

            @pl.when(lane == 0)
            def _():
                outpa[pl.ds(l, L16)] = acca
                outpb[pl.ds(l, L16)] = accb

            na = patha[pl.ds(l * L16 + ia, L16)][0]
            nb = pathb[pl.ds(l * L16 + ib, L16)][0]
            return (na, acca, nb, accb)

        pltpu.sync_copy(outpa, paths_hbm.at[b_img, p_slot])
        pltpu.sync_copy(outpb, paths_hbm.at[b_img, 8 + p_slot])

    @pl.when(p_slot < 7)
    def _():
        abase = (p_slot + 1) * 64 - BW - SH
        hv = pltpu.async_copy(gm_hbm.at[b_img, :, pl.ds(abase, L16)],
                              bandv2, semv)
        hh = pltpu.async_copy(gm_hbm.at[b_img, pl.ds(abase, L16), :],
                              bandh2, semh)
        with jax.named_scope("relay"):
            hh.wait()

            @plsc.parallel_loop(0, H, unroll=8)
            def relay(t):
                bandt[pl.ds(t * L16 + t // 32, L16)] = \
                    bandh2[t // 32, pl.ds(lax.rem(t, 32) * L16, L16)]

        with jax.named_scope("dp"):
            hv.wait()
            fused_dp()


def _label_body(paths_ref, out_ref):
    iw = lax.broadcasted_iota(jnp.int32, (H, W), 1)
    ih = lax.broadcasted_iota(jnp.int32, (H, W), 0)
    acc_v = jnp.zeros((H, W), jnp.int32)
    acc_h = jnp.zeros((H, W), jnp.int32)
    for p in range(7):
        vp = paths_ref[0, p, :]
        acc_v += (vp[:, None] <= iw).astype(jnp.int32)
    for q in range(7):
        hq = paths_ref[0, 8 + q, :]
        acc_h += (hq[None, :] <= ih).astype(jnp.int32)
    out_ref[0] = acc_v + NSEG * acc_h


@jax.jit
def _run(gm):
    B = gm.shape[0]
    mesh = plsc.VectorSubcoreMesh(
        core_axis_name="c", subcore_axis_name="s", num_cores=2,
        num_subcores=16)
    sc_call = pl.kernel(
        _sc_dp_body,
        out_type=jax.ShapeDtypeStruct((4, L16, H), jnp.int32),
        mesh=mesh,
        scratch_types=[
            pltpu.VMEM((H, L16), jnp.float32),
            pltpu.VMEM((L16, H), jnp.float32),
            pltpu.VMEM((L16 * (H + 1),), jnp.float32),
            pltpu.VMEM((H * L16 + L16,), jnp.int32),
            pltpu.VMEM((H * L16 + L16,), jnp.int32),
            pltpu.VMEM((48,), jnp.float32),
            pltpu.VMEM((H,), jnp.int32),
            pltpu.VMEM((H,), jnp.int32),
            pltpu.SemaphoreType.DMA,
            pltpu.SemaphoreType.DMA,
        ],
        compiler_params=pltpu.CompilerParams(use_tc_tiling_on_sc=False,
                                             needs_layout_passes=False),
    )
    paths = sc_call(gm)

    out = pl.pallas_call(
        _label_body,
        grid=(B,),
        in_specs=[pl.BlockSpec((1, 2 * 8, H), lambda b: (b, 0, 0))],
        out_specs=pl.BlockSpec((1, H, W), lambda b: (b, 0, 0)),
        out_shape=jax.ShapeDtypeStruct((B, H, W), jnp.int32),
    )(paths)
    return out


def kernel(grad_map, segmentation_mask, band_width):
    del segmentation_mask, band_width
    return _run(grad_map[:, 0])


# speedup vs baseline: 1.5127x; 1.0573x over previous
import jax
import jax.numpy as jnp
from jax import lax
from jax.experimental import pallas as pl
from jax.experimental.pallas import tpu as pltpu
from jax.experimental.pallas import tpu_sc as plsc

H = 512
W = 512
NPOS = 11
BW = 5
SH = 3
NSEG = 8
L16 = 16
NITEMS = 64
INF = float("inf")


def _vgather(x, idx):
    dnums = lax.GatherDimensionNumbers(
        offset_dims=(), collapsed_slice_dims=(0,), start_index_map=(0,))
    return lax.gather(x, idx[:, None], dnums, (1,),
                      mode=lax.GatherScatterMode.PROMISE_IN_BOUNDS)


def _sc_dp_body(gm_hbm, paths_hbm, bandv2, bandh2, bandt, patha, pathb,
                cost_v, outpa, outpb, semv, semh):
    cid = lax.axis_index("c")
    sid = lax.axis_index("s")
    wid = sid * 2 + cid
    iota = lax.broadcasted_iota(jnp.int32, (L16,), 0)
    shl = jnp.maximum(iota - 1, 0)
    shr = jnp.minimum(iota + 1, L16 - 1)
    zero16 = jnp.zeros((L16,), jnp.int32)
    in_band = (iota >= SH) & (iota < SH + NPOS)
    b_img = wid // 8
    p_slot = lax.rem(wid, 8)
    HP = H + 1

    def fused_dp():
        abase = (p_slot + 1) * 64 - BW - SH
        col0 = iota * HP
        cost0a = jnp.where(in_band, -bandv2[0], INF)
        cost0b = jnp.where(in_band, -plsc.load_gather(bandt, [col0]), INF)

        @plsc.parallel_loop(1, H, carry=(cost0a, cost0b), unroll=8)
        def fwd(l, carry):
            ca, cb = carry
            out = []
            for cost, path_ref in ((ca, patha), (cb, pathb)):
                a = _vgather(cost, shl)
                c = _vgather(cost, shr)
                m = jnp.minimum(jnp.minimum(a, cost), c)
                take_l = (a <= cost) & (a <= c)
                take_m = cost <= c
                delta = jnp.where(take_l, -1, jnp.where(take_m, 0, 1))
                path_ref[pl.ds(l * L16, L16)] = iota + delta.astype(jnp.int32)
                out.append(m)
            na = jnp.where(in_band, out[0] - bandv2[l], INF)
            nb = jnp.where(in_band,
                           out[1] - plsc.load_gather(bandt, [col0 + l]), INF)
            return (na, nb)

        fa, fb = fwd
        cost_v[pl.ds(0, L16)] = fa
        cost_v[pl.ds(L16, L16)] = fb

        def amin(j, carry):
            best, bidx = carry
            c = cost_v[pl.ds(j, L16)][0]
            pred = c < best
            return (jnp.where(pred, c, best),
                    jnp.where(pred, j, bidx))

        _, ia0 = lax.fori_loop(SH, SH + NPOS, amin, (INF, jnp.int32(SH)))
        _, ib0 = lax.fori_loop(L16 + SH, L16 + SH + NPOS, amin,
                               (INF, jnp.int32(L16 + SH)))
        ib0 = ib0 - L16

        @plsc.parallel_loop(0, H, carry=(ia0, zero16, ib0, zero16), unroll=8)
        def bwd(t, carry):
            ia, acca, ib, accb = carry
            l = (H - 1) - t
            lane = lax.rem(l, L16)
            acca = jnp.where(iota == lane, abase + ia, acca)
            accb = jnp.where(iota == lane, abase + ib, accb)

            @pl.when(lane == 0)
            def _():
                outpa[pl.ds(l, L16)] = acca
                outpb[pl.ds(l, L16)] = accb

            na = patha[pl.ds(l * L16 + ia, L16)][0]
            nb = pathb[pl.ds(l * L16 + ib, L16)][0]
            return (na, acca, nb, accb)

        pltpu.sync_copy(outpa, paths_hbm.at[b_img, p_slot])
        pltpu.sync_copy(outpb, paths_hbm.at[b_img, 8 + p_slot])

    @pl.when(p_slot < 7)
    def _():
        abase = (p_slot + 1) * 64 - BW - SH
        hv = pltpu.async_copy(gm_hbm.at[b_img, :, pl.ds(abase, L16)],
                              bandv2, semv)
        hh = pltpu.async_copy(gm_hbm.at[b_img, pl.ds(abase, L16), :],
                              bandh2, semh)
        with jax.named_scope("relay"):
            hh.wait()

            @plsc.parallel_loop(0, H, unroll=8)
            def relay(t):
                bandt[pl.ds(t * L16 + t // 32, L16)] = \
                    bandh2[t // 32, pl.ds(lax.rem(t, 32) * L16, L16)]

        with jax.named_scope("dp"):
            hv.wait()
            fused_dp()


def _label_body(paths_ref, out_ref):
    iw = lax.broadcasted_iota(jnp.int16, (H, W), 1)
    ih = lax.broadcasted_iota(jnp.int16, (H, W), 0)
    acc_v = jnp.zeros((H, W), jnp.int16)
    acc_h = jnp.zeros((H, W), jnp.int16)
    for p in range(7):
        vp = paths_ref[0, p, :].astype(jnp.int16)
        acc_v += (vp[:, None] <= iw).astype(jnp.int16)
    for q in range(7):
        hq = paths_ref[0, 8 + q, :].astype(jnp.int16)
        acc_h += (hq[None, :] <= ih).astype(jnp.int16)
    out_ref[0] = (acc_v + NSEG * acc_h).astype(jnp.int32)


@jax.jit
def _run(gm):
    B = gm.shape[0]
    mesh = plsc.VectorSubcoreMesh(
        core_axis_name="c", subcore_axis_name="s", num_cores=2,
        num_subcores=16)
    sc_call = pl.kernel(
        _sc_dp_body,
        out_type=jax.ShapeDtypeStruct((4, L16, H), jnp.int32),
        mesh=mesh,
        scratch_types=[
            pltpu.VMEM((H, L16), jnp.float32),
            pltpu.VMEM((L16, H), jnp.float32),
            pltpu.VMEM((L16 * (H + 1),), jnp.float32),
            pltpu.VMEM((H * L16 + L16,), jnp.int32),
            pltpu.VMEM((H * L16 + L16,), jnp.int32),
            pltpu.VMEM((48,), jnp.float32),
            pltpu.VMEM((H,), jnp.int32),
            pltpu.VMEM((H,), jnp.int32),
            pltpu.SemaphoreType.DMA,
            pltpu.SemaphoreType.DMA,
        ],
        compiler_params=pltpu.CompilerParams(use_tc_tiling_on_sc=False,
                                             needs_layout_passes=False),
    )
    paths = sc_call(gm)

    out = pl.pallas_call(
        _label_body,
        grid=(B,),
        in_specs=[pl.BlockSpec((1, 2 * 8, H), lambda b: (b, 0, 0))],
        out_specs=pl.BlockSpec((1, H, W), lambda b: (b, 0, 0)),
        out_shape=jax.ShapeDtypeStruct((B, H, W), jnp.int32),
    )(paths)
    return out


def kernel(grad_map, segmentation_mask, band_width):
    del segmentation_mask, band_width
    return _run(grad_map[:, 0])


# speedup vs baseline: 1.6169x; 1.0689x over previous
import jax
import jax.numpy as jnp
from jax import lax
from jax.experimental import pallas as pl
from jax.experimental.pallas import tpu as pltpu
from jax.experimental.pallas import tpu_sc as plsc

H = 512
W = 512
NPOS = 11
BW = 5
SH = 3
NSEG = 8
L16 = 16
NITEMS = 64
INF = float("inf")


def _vgather(x, idx):
    dnums = lax.GatherDimensionNumbers(
        offset_dims=(), collapsed_slice_dims=(0,), start_index_map=(0,))
    return lax.gather(x, idx[:, None], dnums, (1,),
                      mode=lax.GatherScatterMode.PROMISE_IN_BOUNDS)


def _sc_dp_body(gm_hbm, paths_hbm, bandv2, bandh2, bandt, patha, pathb,
                path2a, path2b, cost_v, outpa, outpb, semv, semh):
    cid = lax.axis_index("c")
    sid = lax.axis_index("s")
    wid = sid * 2 + cid
    iota = lax.broadcasted_iota(jnp.int32, (L16,), 0)
    shl = jnp.maximum(iota - 1, 0)
    shr = jnp.minimum(iota + 1, L16 - 1)
    zero16 = jnp.zeros((L16,), jnp.int32)
    in_band = (iota >= SH) & (iota < SH + NPOS)
    b_img = wid // 8
    p_slot = lax.rem(wid, 8)
    HP = H + 1

    def fused_dp():
        abase = (p_slot + 1) * 64 - BW - SH
        col0 = iota * HP
        cost0a = jnp.where(in_band, -bandv2[0], INF)
        cost0b = jnp.where(in_band, -plsc.load_gather(bandt, [col0]), INF)

        @plsc.parallel_loop(1, H, carry=(cost0a, cost0b, iota, iota),
                            unroll=8)
        def fwd(l, carry):
            ca, cb, p1a, p1b = carry
            out = []
            for cost, p1p, path_ref, path2_ref in (
                    (ca, p1a, patha, path2a), (cb, p1b, pathb, path2b)):
                a = _vgather(cost, shl)
                c = _vgather(cost, shr)
                m = jnp.minimum(jnp.minimum(a, cost), c)
                take_l = (a <= cost) & (a <= c)
                take_m = cost <= c
                delta = jnp.where(take_l, -1, jnp.where(take_m, 0, 1))
                p1 = iota + delta.astype(jnp.int32)
                path_ref[pl.ds(l * L16, L16)] = p1
                path2_ref[pl.ds(l * L16, L16)] = _vgather(p1p, p1)
                out.append((m, p1))
            na = jnp.where(in_band, out[0][0] - bandv2[l], INF)
            nb = jnp.where(in_band,
                           out[1][0] - plsc.load_gather(bandt, [col0 + l]),
                           INF)
            return (na, nb, out[0][1], out[1][1])

        fa, fb = fwd[0], fwd[1]
        cost_v[pl.ds(0, L16)] = fa
        cost_v[pl.ds(L16, L16)] = fb

        def amin(j, carry):
            best, bidx = carry
            c = cost_v[pl.ds(j, L16)][0]
            pred = c < best
            return (jnp.where(pred, c, best),
                    jnp.where(pred, j, bidx))

        _, ia0 = lax.fori_loop(SH, SH + NPOS, amin, (INF, jnp.int32(SH)))
        _, ib0 = lax.fori_loop(L16 + SH, L16 + SH + NPOS, amin,
                               (INF, jnp.int32(L16 + SH)))
        ib0 = ib0 - L16

        @plsc.parallel_loop(0, H // 2, carry=(ia0, zero16, ib0, zero16),
                            unroll=8)
        def bwd(t, carry):
            ia, acca, ib, accb = carry
            l = (H - 1) - 2 * t
            lane = lax.rem(l, L16)
            iam1 = patha[pl.ds(l * L16 + ia, L16)][0]
            ibm1 = pathb[pl.ds(l * L16 + ib, L16)][0]
            iam2 = path2a[pl.ds(l * L16 + ia, L16)][0]
            ibm2 = path2b[pl.ds(l * L16 + ib, L16)][0]
            acca = jnp.where(iota == lane, abase + ia,
                             jnp.where(iota == lane - 1, abase + iam1, acca))
            accb = jnp.where(iota == lane, abase + ib,
                             jnp.where(iota == lane - 1, abase + ibm1, accb))

            @pl.when(lane == 1)
            def _():
                outpa[pl.ds(l - 1, L16)] = acca
                outpb[pl.ds(l - 1, L16)] = accb

            return (iam2, acca, ibm2, accb)

        pltpu.sync_copy(outpa, paths_hbm.at[b_img, p_slot])
        pltpu.sync_copy(outpb, paths_hbm.at[b_img, 8 + p_slot])

    @pl.when(p_slot < 7)
    def _():
        abase = (p_slot + 1) * 64 - BW - SH
        hh = pltpu.async_copy(gm_hbm.at[b_img, pl.ds(abase, L16), :],
                              bandh2, semh)
        hv = pltpu.async_copy(gm_hbm.at[b_img, :, pl.ds(abase, L16)],
                              bandv2, semv)
        with jax.named_scope("relay"):
            hh.wait()

            @plsc.parallel_loop(0, H, unroll=8)
            def relay(t):
                bandt[pl.ds(t * L16 + t // 32, L16)] = \
                    bandh2[t // 32, pl.ds(lax.rem(t, 32) * L16, L16)]

        with jax.named_scope("dp"):
            hv.wait()
            fused_dp()


def _label_body(paths_ref, out_ref):
    iw = lax.broadcasted_iota(jnp.int16, (H, W), 1)
    ih = lax.broadcasted_iota(jnp.int16, (H, W), 0)
    acc_v = jnp.zeros((H, W), jnp.int16)
    acc_h = jnp.zeros((H, W), jnp.int16)
    for p in range(7):
        vp = paths_ref[0, p, :].astype(jnp.int16)
        acc_v += (vp[:, None] <= iw).astype(jnp.int16)
    for q in range(7):
        hq = paths_ref[0, 8 + q, :].astype(jnp.int16)
        acc_h += (hq[None, :] <= ih).astype(jnp.int16)
    out_ref[0] = (acc_v + NSEG * acc_h).astype(jnp.int32)


@jax.jit
def _run(gm):
    B = gm.shape[0]
    mesh = plsc.VectorSubcoreMesh(
        core_axis_name="c", subcore_axis_name="s", num_cores=2,
        num_subcores=16)
    sc_call = pl.kernel(
        _sc_dp_body,
        out_type=jax.ShapeDtypeStruct((4, L16, H), jnp.int32),
        mesh=mesh,
        scratch_types=[
            pltpu.VMEM((H, L16), jnp.float32),
            pltpu.VMEM((L16, H), jnp.float32),
            pltpu.VMEM((L16 * (H + 1),), jnp.float32),
            pltpu.VMEM((H * L16 + L16,), jnp.int32),
            pltpu.VMEM((H * L16 + L16,), jnp.int32),
            pltpu.VMEM((H * L16 + L16,), jnp.int32),
            pltpu.VMEM((H * L16 + L16,), jnp.int32),
            pltpu.VMEM((48,), jnp.float32),
            pltpu.VMEM((H,), jnp.int32),
            pltpu.VMEM((H,), jnp.int32),
            pltpu.SemaphoreType.DMA,
            pltpu.SemaphoreType.DMA,
        ],
        compiler_params=pltpu.CompilerParams(use_tc_tiling_on_sc=False,
                                             needs_layout_passes=False),
    )
    paths = sc_call(gm)

    out = pl.pallas_call(
        _label_body,
        grid=(B,),
        in_specs=[pl.BlockSpec((1, 2 * 8, H), lambda b: (b, 0, 0))],
        out_specs=pl.BlockSpec((1, H, W), lambda b: (b, 0, 0)),
        out_shape=jax.ShapeDtypeStruct((B, H, W), jnp.int32),
    )(paths)
    return out


def kernel(grad_map, segmentation_mask, band_width):
    del segmentation_mask, band_width
    return _run(grad_map[:, 0])


# speedup vs baseline: 1.6185x; 1.0010x over previous
import jax
import jax.numpy as jnp
from jax import lax
from jax.experimental import pallas as pl
from jax.experimental.pallas import tpu as pltpu
from jax.experimental.pallas import tpu_sc as plsc

H = 512
W = 512
NPOS = 11
BW = 5
SH = 3
NSEG = 8
L16 = 16
NITEMS = 64
INF = float("inf")


def _vgather(x, idx):
    dnums = lax.GatherDimensionNumbers(
        offset_dims=(), collapsed_slice_dims=(0,), start_index_map=(0,))
    return lax.gather(x, idx[:, None], dnums, (1,),
                      mode=lax.GatherScatterMode.PROMISE_IN_BOUNDS)


def _sc_dp_body(gm_hbm, paths_hbm, bandv2, bandh2, bandt, patha, pathb,
                path2a, path2b, cost_v, outpa, outpb, semv, semh):
    cid = lax.axis_index("c")
    sid = lax.axis_index("s")
    wid = sid * 2 + cid
    iota = lax.broadcasted_iota(jnp.int32, (L16,), 0)
    shl = jnp.maximum(iota - 1, 0)
    shr = jnp.minimum(iota + 1, L16 - 1)
    zero16 = jnp.zeros((L16,), jnp.int32)
    in_band = (iota >= SH) & (iota < SH + NPOS)
    b_img = wid // 8
    p_slot = lax.rem(wid, 8)
    HP = H + 1

    def fused_dp():
        abase = (p_slot + 1) * 64 - BW - SH
        col0 = iota * HP
        cost0a = jnp.where(in_band, -bandv2[0], INF)
        cost0b = jnp.where(in_band, -plsc.load_gather(bandt, [col0]), INF)

        @plsc.parallel_loop(1, H, carry=(cost0a, cost0b, iota, iota),
                            unroll=16)
        def fwd(l, carry):
            ca, cb, p1a, p1b = carry
            out = []
            for cost, p1p, path_ref, path2_ref in (
                    (ca, p1a, patha, path2a), (cb, p1b, pathb, path2b)):
                a = _vgather(cost, shl)
                c = _vgather(cost, shr)
                m = jnp.minimum(jnp.minimum(a, cost), c)
                take_l = (a <= cost) & (a <= c)
                take_m = cost <= c
                delta = jnp.where(take_l, -1, jnp.where(take_m, 0, 1))
                p1 = iota + delta.astype(jnp.int32)
                path_ref[pl.ds(l * L16, L16)] = p1
                path2_ref[pl.ds(l * L16, L16)] = _vgather(p1p, p1)
                out.append((m, p1))
            na = jnp.where(in_band, out[0][0] - bandv2[l], INF)
            nb = jnp.where(in_band,
                           out[1][0] - plsc.load_gather(bandt, [col0 + l]),
                           INF)
            return (na, nb, out[0][1], out[1][1])

        fa, fb = fwd[0], fwd[1]
        cost_v[pl.ds(0, L16)] = fa
        cost_v[pl.ds(L16, L16)] = fb

        def amin(j, carry):
            best, bidx = carry
            c = cost_v[pl.ds(j, L16)][0]
            pred = c < best
            return (jnp.where(pred, c, best),
                    jnp.where(pred, j, bidx))

        _, ia0 = lax.fori_loop(SH, SH + NPOS, amin, (INF, jnp.int32(SH)))
        _, ib0 = lax.fori_loop(L16 + SH, L16 + SH + NPOS, amin,
                               (INF, jnp.int32(L16 + SH)))
        ib0 = ib0 - L16

        @plsc.parallel_loop(0, H // 2, carry=(ia0, zero16, ib0, zero16),
                            unroll=16)
        def bwd(t, carry):
            ia, acca, ib, accb = carry
            l = (H - 1) - 2 * t
            lane = lax.rem(l, L16)
            iam1 = patha[pl.ds(l * L16 + ia, L16)][0]
            ibm1 = pathb[pl.ds(l * L16 + ib, L16)][0]
            iam2 = path2a[pl.ds(l * L16 + ia, L16)][0]
            ibm2 = path2b[pl.ds(l * L16 + ib, L16)][0]
            acca = jnp.where(iota == lane, abase + ia,
                             jnp.where(iota == lane - 1, abase + iam1, acca))
            accb = jnp.where(iota == lane, abase + ib,
                             jnp.where(iota == lane - 1, abase + ibm1, accb))

            @pl.when(lane == 1)
            def _():
                outpa[pl.ds(l - 1, L16)] = acca
                outpb[pl.ds(l - 1, L16)] = accb

            return (iam2, acca, ibm2, accb)

        pltpu.sync_copy(outpa, paths_hbm.at[b_img, p_slot])
        pltpu.sync_copy(outpb, paths_hbm.at[b_img, 8 + p_slot])

    @pl.when(p_slot < 7)
    def _():
        abase = (p_slot + 1) * 64 - BW - SH
        hh = pltpu.async_copy(gm_hbm.at[b_img, pl.ds(abase, L16), :],
                              bandh2, semh)
        hv = pltpu.async_copy(gm_hbm.at[b_img, :, pl.ds(abase, L16)],
                              bandv2, semv)
        hh.wait()

        @plsc.parallel_loop(0, H, unroll=8)
        def relay(t):
            bandt[pl.ds(t * L16 + t // 32, L16)] = \
                bandh2[t // 32, pl.ds(lax.rem(t, 32) * L16, L16)]

        hv.wait()
        fused_dp()


def _label_body(paths_ref, out_ref):
    iw = lax.broadcasted_iota(jnp.int16, (H, W), 1)
    ih = lax.broadcasted_iota(jnp.int16, (H, W), 0)
    acc_v = jnp.zeros((H, W), jnp.int16)
    acc_h = jnp.zeros((H, W), jnp.int16)
    for p in range(7):
        vp = paths_ref[0, p, :].astype(jnp.int16)
        acc_v += (vp[:, None] <= iw).astype(jnp.int16)
    for q in range(7):
        hq = paths_ref[0, 8 + q, :].astype(jnp.int16)
        acc_h += (hq[None, :] <= ih).astype(jnp.int16)
    out_ref[0] = (acc_v + NSEG * acc_h).astype(jnp.int32)


@jax.jit
def _run(gm):
    B = gm.shape[0]
    mesh = plsc.VectorSubcoreMesh(
        core_axis_name="c", subcore_axis_name="s", num_cores=2,
        num_subcores=16)
    sc_call = pl.kernel(
        _sc_dp_body,
        out_type=jax.ShapeDtypeStruct((4, L16, H), jnp.int32),
        mesh=mesh,
        scratch_types=[
            pltpu.VMEM((H, L16), jnp.float32),
            pltpu.VMEM((L16, H), jnp.float32),
            pltpu.VMEM((L16 * (H + 1),), jnp.float32),
            pltpu.VMEM((H * L16 + L16,), jnp.int32),
            pltpu.VMEM((H * L16 + L16,), jnp.int32),
            pltpu.VMEM((H * L16 + L16,), jnp.int32),
            pltpu.VMEM((H * L16 + L16,), jnp.int32),
            pltpu.VMEM((48,), jnp.float32),
            pltpu.VMEM((H,), jnp.int32),
            pltpu.VMEM((H,), jnp.int32),
            pltpu.SemaphoreType.DMA,
            pltpu.SemaphoreType.DMA,
        ],
        compiler_params=pltpu.CompilerParams(use_tc_tiling_on_sc=False,
                                             needs_layout_passes=False),
    )
    paths = sc_call(gm)

    out = pl.pallas_call(
        _label_body,
        grid=(B,),
        in_specs=[pl.BlockSpec((1, 2 * 8, H), lambda b: (b, 0, 0))],
        out_specs=pl.BlockSpec((1, H, W), lambda b: (b, 0, 0)),
        out_shape=jax.ShapeDtypeStruct((B, H, W), jnp.int32),
    )(paths)
    return out


def kernel(grad_map, segmentation_mask, band_width):
    del segmentation_mask, band_width
    return _run(grad_map[:, 0])
